# Initial kernel scaffold; baseline (speedup 1.0000x reference)
import jax
import jax.numpy as jnp
from jax.experimental import pallas as pl


def kernel(flat_values, cu_seqlens):
    raise NotImplementedError("write your pallas kernel here")


# speedup vs baseline: 3.9680x; 3.9680x over previous
import functools

import jax
import jax.numpy as jnp
from jax import lax
from jax.experimental import pallas as pl
from jax.experimental.pallas import tpu as pltpu
from jax.experimental.pallas import tpu_sc as plsc

_B = 16
_S = 2048
_D = 256
_T = 16384

_NC = 2
_NS = 16
_NW = _NC * _NS
_CH = (_B * _S) // _NW
_CPB = _S // _CH
_P = 64
_NP = _CH // _P
_NV = _D // 16


def _body(flat_hbm, cu_hbm, out_hbm, cu_v, idx_v, zbuf, buf, sem):
    cid = lax.axis_index("c")
    sid = lax.axis_index("s")
    wid = sid * _NC + cid
    b = wid // _CPB
    s0 = (wid % _CPB) * _CH

    pltpu.sync_copy(cu_hbm, cu_v)
    lane = lax.iota(jnp.int32, 16)
    cu_vec = cu_v[pl.ds(0, 16)]
    ivec = jnp.minimum(b + lane, _B - 1)
    gathered = cu_vec.at[ivec].get(mode="promise_in_bounds")
    start = gathered[0]
    end = jnp.where(b + 1 >= _B, _T, gathered[1])
    lim = jnp.minimum(end - start, _S)
    v = jnp.clip(lim - s0, 0, _CH)
    nfull = v // _P
    vp = v - nfull * _P

    zero16 = jnp.zeros((16,), jnp.float32)

    def zloop(i, carry):
        r = i // _NV
        c = (i % _NV) * 16
        zbuf[r, pl.ds(c, 16)] = zero16
        return carry

    lax.fori_loop(0, _P * _NV, zloop, 0)

    base = start + s0

    def piece(p, carry):
        dst = out_hbm.at[b, pl.ds(s0 + p * _P, _P), :]
        src = base + p * _P
        is_full = p < nfull
        is_part = jnp.logical_and(p == nfull, vp > 0)

        def gather_piece():
            for k in range(_P // 16):
                idx_v[pl.ds(k * 16, 16)] = jnp.minimum(
                    src + k * 16 + lane, _T - 1
                )
            pltpu.async_copy(flat_hbm.at[idx_v], buf, sem).wait()

        @pl.when(is_full)
        def _full():
            gather_piece()
            pltpu.sync_copy(buf, dst)

        @pl.when(is_part)
        def _part():
            gather_piece()

            def tloop(i, c2):
                r = vp + i // _NV
                c = (i % _NV) * 16
                buf[r, pl.ds(c, 16)] = zero16
                return c2

            lax.fori_loop(0, (_P - vp) * _NV, tloop, 0)
            pltpu.sync_copy(buf, dst)

        @pl.when(jnp.logical_not(jnp.logical_or(is_full, is_part)))
        def _zero():
            pltpu.sync_copy(zbuf, dst)

        return carry

    lax.fori_loop(0, _NP, piece, 0)


_sc_kernel = functools.partial(
    pl.kernel,
    out_type=jax.ShapeDtypeStruct((_B, _S, _D), jnp.float32),
    mesh=plsc.VectorSubcoreMesh(core_axis_name="c", subcore_axis_name="s"),
    scratch_types=[
        pltpu.VMEM((_B + 1,), jnp.int32),
        pltpu.VMEM((_P,), jnp.int32),
        pltpu.VMEM((_P, _D), jnp.float32),
        pltpu.VMEM((_P, _D), jnp.float32),
        pltpu.SemaphoreType.DMA,
    ],
)(_body)


@jax.jit
def kernel(flat_values, cu_seqlens):
    return _sc_kernel(flat_values, cu_seqlens)


# speedup vs baseline: 4.1340x; 1.0418x over previous
import functools

import jax
import jax.numpy as jnp
from jax import lax
from jax.experimental import pallas as pl
from jax.experimental.pallas import tpu as pltpu
from jax.experimental.pallas import tpu_sc as plsc

_B = 16
_S = 2048
_D = 256
_T = 16384

_NC = 2
_NS = 16
_NW = _NC * _NS
_CH = (_B * _S) // _NW
_CPB = _S // _CH
_P = 128
_NP = _CH // _P
_NV = _D // 16


def _body(flat_hbm, cu_hbm, out_hbm, cu_v, idx0, idx1, zbuf, buf0, buf1,
          gsem0, gsem1, ssem0, ssem1, zsem):
    idxs = (idx0, idx1)
    bufs = (buf0, buf1)
    gsems = (gsem0, gsem1)
    ssems = (ssem0, ssem1)

    cid = lax.axis_index("c")
    sid = lax.axis_index("s")
    wid = sid * _NC + cid
    b = wid // _CPB
    s0 = (wid % _CPB) * _CH

    pltpu.sync_copy(cu_hbm, cu_v)
    lane = lax.iota(jnp.int32, 16)
    cu_vec = cu_v[pl.ds(0, 16)]
    ivec = jnp.minimum(b + lane, _B - 1)
    gathered = cu_vec.at[ivec].get(mode="promise_in_bounds")
    start = gathered[0]
    end = jnp.where(b + 1 >= _B, _T, gathered[1])
    lim = jnp.minimum(end - start, _S)
    v = jnp.clip(lim - s0, 0, _CH)
    nfull = v // _P
    vp = v - nfull * _P
    nvalid = nfull + jnp.where(vp > 0, 1, 0)
    nzero = _NP - nvalid

    zero16 = jnp.zeros((16,), jnp.float32)

    def zloop(i, carry):
        r = i // _NV
        c = (i % _NV) * 16
        zbuf[r, pl.ds(c, 16)] = zero16
        return carry

    lax.fori_loop(0, _P * _NV, zloop, 0)

    base = start + s0

    def dst(p):
        return out_hbm.at[b, pl.ds(s0 + p * _P, _P), :]

    for p in range(_NP):
        @pl.when(p >= nvalid)
        def _z(p=p):
            pltpu.make_async_copy(zbuf, dst(p), zsem).start()

    def fill_idx(i_ref, p):
        for k in range(_P // 16):
            i_ref[pl.ds(k * 16, 16)] = jnp.minimum(
                base + p * _P + k * 16 + lane, _T - 1
            )

    def gather(p, slot):
        return pltpu.make_async_copy(flat_hbm.at[idxs[slot]], bufs[slot],
                                     gsems[slot])

    def store(p, slot):
        return pltpu.make_async_copy(bufs[slot], dst(p), ssems[slot])

    @pl.when(nvalid > 0)
    def _pro():
        fill_idx(idxs[0], 0)
        gather(0, 0).start()

    for p in range(_NP):
        slot = p % 2
        nslot = (p + 1) % 2

        @pl.when(p < nvalid)
        def _piece(p=p, slot=slot, nslot=nslot):
            gather(p, slot).wait()

            @pl.when(p + 1 < nvalid)
            def _prefetch():
                if p >= 1:
                    store(p - 1, nslot).wait()
                fill_idx(idxs[nslot], p + 1)
                gather(p + 1, nslot).start()

            @pl.when(jnp.logical_and(p == nfull, vp > 0))
            def _tail():
                def tloop(i, c2):
                    r = vp + i // _NV
                    c = (i % _NV) * 16
                    bufs[slot][r, pl.ds(c, 16)] = zero16
                    return c2

                lax.fori_loop(0, (_P - vp) * _NV, tloop, 0)

            store(p, slot).start()

    @pl.when(nvalid >= 1)
    def _d1():
        pltpu.make_async_copy(bufs[0], dst(0), ssems[0]).wait()

    @pl.when(nvalid >= 2)
    def _d2():
        pltpu.make_async_copy(bufs[1], dst(0), ssems[1]).wait()

    def zdrain(i, carry):
        pltpu.make_async_copy(zbuf, dst(0), zsem).wait()
        return carry

    lax.fori_loop(0, nzero, zdrain, 0)


_sc_kernel = functools.partial(
    pl.kernel,
    out_type=jax.ShapeDtypeStruct((_B, _S, _D), jnp.float32),
    mesh=plsc.VectorSubcoreMesh(core_axis_name="c", subcore_axis_name="s"),
    scratch_types=[
        pltpu.VMEM((_B + 1,), jnp.int32),
        pltpu.VMEM((_P,), jnp.int32),
        pltpu.VMEM((_P,), jnp.int32),
        pltpu.VMEM((_P, _D), jnp.float32),
        pltpu.VMEM((_P, _D), jnp.float32),
        pltpu.VMEM((_P, _D), jnp.float32),
        pltpu.SemaphoreType.DMA,
        pltpu.SemaphoreType.DMA,
        pltpu.SemaphoreType.DMA,
        pltpu.SemaphoreType.DMA,
        pltpu.SemaphoreType.DMA,
    ],
)(_body)


@jax.jit
def kernel(flat_values, cu_seqlens):
    return _sc_kernel(flat_values, cu_seqlens)
